# baseline (speedup 1.0000x reference)
import functools

import jax
import jax.numpy as jnp
from jax import lax
from jax.experimental import pallas as pl
from jax.experimental.pallas import tpu as pltpu
from jax.experimental.pallas import tpu_sc as plsc

D = 256
K = 512
BLK = 1024
PAD = -1
CW = 0.25
NCHUNK = 4


def _tc_body(flat_ref, mask_ref, embed_ref, iota_ref, idxg_ref, idxo_ref,
             loss_ref):
    i = pl.program_id(0)
    f = flat_ref[...]
    e = embed_ref[...]
    fg = lax.dot_general(f, e, (((1,), (1,)), ((), ())),
                         preferred_element_type=jnp.float32)
    f2 = jnp.sum(f * f, axis=1, keepdims=True)
    e2 = jnp.sum(e * e, axis=1)
    dist = f2 - 2.0 * fg + e2[None, :]
    md = jnp.min(dist, axis=1, keepdims=True)
    idxf = jnp.min(jnp.where(dist <= md, iota_ref[...], float(K)), axis=1)
    idx2 = idxf.astype(jnp.int32).reshape(BLK // 128, 128)
    m = mask_ref[...] > 0
    idxg_ref[...] = jnp.where(m, idx2, K)
    idxo_ref[...] = jnp.where(m, idx2, PAD)
    s = jnp.sum(md)

    @pl.when(i == 0)
    def _():
        loss_ref[...] = jnp.zeros_like(loss_ref)

    loss_ref[...] += s


def _tc_stage(flat, mask2d, embed):
    rows = flat.shape[0]
    nblk = rows // BLK
    sub = BLK // 128
    iota = jnp.arange(K, dtype=jnp.float32).reshape(1, K)
    return pl.pallas_call(
        _tc_body,
        grid=(nblk,),
        in_specs=[
            pl.BlockSpec((BLK, D), lambda i: (i, 0)),
            pl.BlockSpec((sub, 128), lambda i: (i, 0)),
            pl.BlockSpec((K, D), lambda i: (0, 0)),
            pl.BlockSpec((1, K), lambda i: (0, 0)),
        ],
        out_specs=[
            pl.BlockSpec((sub, 128), lambda i: (i, 0)),
            pl.BlockSpec((sub, 128), lambda i: (i, 0)),
            pl.BlockSpec((1, 1), lambda i: (0, 0)),
        ],
        out_shape=[
            jax.ShapeDtypeStruct((rows // 128, 128), jnp.int32),
            jax.ShapeDtypeStruct((rows // 128, 128), jnp.int32),
            jax.ShapeDtypeStruct((1, 1), jnp.float32),
        ],
    )(flat, mask2d, embed, iota)


def _sc_gather_into(out_ref, table3, idx2, rows_c, row0):
    info = plsc.get_sparse_core_info()
    ns = info.num_subcores
    half = D // 2
    per_s = rows_c // ns
    nsb = per_s // 128
    mesh = plsc.VectorSubcoreMesh(core_axis_name="c", subcore_axis_name="s")

    @functools.partial(
        pl.kernel,
        mesh=mesh,
        out_type=(),
        scratch_types=[
            pltpu.VMEM(((K + 1) * half,), jnp.float32),
            pltpu.VMEM((per_s,), jnp.int32),
            pltpu.VMEM((2, 128, half), jnp.float32),
            pltpu.SemaphoreType.DMA,
            pltpu.SemaphoreType.DMA,
        ],
    )
    def k(table_hbm, idx_hbm, out_hbm, tab_v, idx_v, stg, sem0, sem1):
        c = lax.axis_index("c")
        s = lax.axis_index("s")
        pltpu.sync_copy(table_hbm.at[c], tab_v)
        pltpu.sync_copy(idx_hbm.at[s], idx_v)
        sems = (sem0, sem1)

        def out_slice(sb):
            return out_hbm.at[pl.ds(row0 + s * per_s + sb * 128, 128),
                              pl.ds(c * half, half)]

        def fill(sb, b):
            def g_body(g, carry2):
                idxv = idx_v[pl.ds(sb * 128 + g * 16, 16)]
                for l in range(16):
                    off = idxv[l] * half
                    r = g * 16 + l
                    for kk in range(half // 16):
                        stg[b, r, pl.ds(kk * 16, 16)] = (
                            tab_v[pl.ds(off + kk * 16, 16)])
                return carry2

            lax.fori_loop(0, 8, g_body, 0)

        def sb2_body(t, carry):
            for b in range(2):
                sb = t * 2 + b

                @pl.when(t > 0)
                def _():
                    pltpu.make_async_copy(
                        stg.at[b], out_slice(sb), sems[b]).wait()

                fill(sb, b)
                pltpu.async_copy(stg.at[b], out_slice(sb), sems[b])
            return carry

        lax.fori_loop(0, nsb // 2, sb2_body, 0)
        for b in range(2):
            pltpu.make_async_copy(
                stg.at[b], out_slice(nsb - 2 + b), sems[b]).wait()

    k(table3, idx2, out_ref)


def kernel(z, phoneme_mask, embed):
    B, N, Dz = z.shape
    rows = B * N
    rows_c = rows // NCHUNK
    flat = z.reshape(rows, Dz)
    mask2d = phoneme_mask.reshape(rows // 128, 128).astype(jnp.int32)
    table = jnp.concatenate([embed, jnp.zeros((1, Dz), jnp.float32)], axis=0)
    table3 = table.reshape(K + 1, 2, Dz // 2).transpose(1, 0, 2).reshape(2, -1)

    out_ref = jax.new_ref(jnp.zeros((rows, Dz), jnp.float32))
    idxos = []
    loss = jnp.zeros((), jnp.float32)
    for ci in range(NCHUNK):
        r0 = ci * rows_c
        fc = flat[r0:r0 + rows_c]
        mc = mask2d[r0 // 128:(r0 + rows_c) // 128]
        idxg, idxo, lossc = _tc_stage(fc, mc, embed)
        idxos.append(idxo)
        loss = loss + lossc[0, 0]
        _sc_gather_into(out_ref, table3, idxg.reshape(16, rows_c // 16),
                        rows_c, r0)

    quant = jax.freeze(out_ref)
    quantize = quant.reshape(B, N, Dz)
    indices = jnp.concatenate(idxos, axis=0).reshape(B, N)
    commit_loss = loss * (CW / (rows * Dz))
    return (quantize, indices, commit_loss)


# speedup vs baseline: 2.1072x; 2.1072x over previous
import functools

import jax
import jax.numpy as jnp
from jax import lax
from jax.experimental import pallas as pl
from jax.experimental.pallas import tpu as pltpu
from jax.experimental.pallas import tpu_sc as plsc

D = 256
K = 512
BLK = 1024
PAD = -1
CW = 0.25


def _tc_body(flat_ref, mask_ref, embed_ref, iota_ref, idxg_ref, idxo_ref,
             loss_ref):
    i = pl.program_id(0)
    f = flat_ref[...]
    e = embed_ref[...]
    fg = lax.dot_general(f, e, (((1,), (1,)), ((), ())),
                         preferred_element_type=jnp.float32)
    f2 = jnp.sum(f * f, axis=1, keepdims=True)
    e2 = jnp.sum(e * e, axis=1)
    dist = f2 - 2.0 * fg + e2[None, :]
    md = jnp.min(dist, axis=1, keepdims=True)
    idxf = jnp.min(jnp.where(dist <= md, iota_ref[...], float(K)), axis=1)
    idx2 = idxf.astype(jnp.int32).reshape(BLK // 128, 128)
    m = mask_ref[...] > 0
    idxg_ref[...] = jnp.where(m, idx2, K)
    idxo_ref[...] = jnp.where(m, idx2, PAD)
    s = jnp.sum(md)

    @pl.when(i == 0)
    def _():
        loss_ref[...] = jnp.zeros_like(loss_ref)

    loss_ref[...] += s


def _tc_stage(flat, mask2d, embed):
    rows = flat.shape[0]
    nblk = rows // BLK
    sub = BLK // 128
    iota = jnp.arange(K, dtype=jnp.float32).reshape(1, K)
    return pl.pallas_call(
        _tc_body,
        grid=(nblk,),
        in_specs=[
            pl.BlockSpec((BLK, D), lambda i: (i, 0)),
            pl.BlockSpec((sub, 128), lambda i: (i, 0)),
            pl.BlockSpec((K, D), lambda i: (0, 0)),
            pl.BlockSpec((1, K), lambda i: (0, 0)),
        ],
        out_specs=[
            pl.BlockSpec((sub, 128), lambda i: (i, 0)),
            pl.BlockSpec((sub, 128), lambda i: (i, 0)),
            pl.BlockSpec((1, 1), lambda i: (0, 0)),
        ],
        out_shape=[
            jax.ShapeDtypeStruct((rows // 128, 128), jnp.int32),
            jax.ShapeDtypeStruct((rows // 128, 128), jnp.int32),
            jax.ShapeDtypeStruct((1, 1), jnp.float32),
        ],
    )(flat, mask2d, embed, iota)


def _sc_gather(table3, idx2, rows):
    info = plsc.get_sparse_core_info()
    ns = info.num_subcores
    half = D // 2
    per_s = rows // ns
    mesh = plsc.VectorSubcoreMesh(core_axis_name="c", subcore_axis_name="s")

    @functools.partial(
        pl.kernel,
        mesh=mesh,
        out_type=jax.ShapeDtypeStruct((rows, D), jnp.float32),
        scratch_types=[
            pltpu.VMEM(((K + 1) * half,), jnp.float32),
            pltpu.VMEM((per_s,), jnp.int32),
            pltpu.SemaphoreType.DMA,
        ],
    )
    def k(table_hbm, idx_hbm, out_hbm, tab_v, idx_v, sem0):
        c = lax.axis_index("c")
        s = lax.axis_index("s")
        pltpu.sync_copy(table_hbm.at[c], tab_v)
        pltpu.sync_copy(idx_hbm.at[s], idx_v)
        base = s * per_s

        def g_body(g, carry):
            idxv = idx_v[pl.ds(g * 16, 16)]
            r0 = base + g * 16
            for l in range(16):
                off = idxv[l] * half
                pltpu.async_copy(
                    tab_v.at[pl.ds(off, half)],
                    out_hbm.at[r0 + l, pl.ds(c * half, half)],
                    sem0)
            return carry

        lax.fori_loop(0, per_s // 16, g_body, 0)

        def w_body(g, carry):
            for l in range(16):
                pltpu.make_async_copy(
                    tab_v.at[pl.ds(0, half)],
                    out_hbm.at[base + l, pl.ds(c * half, half)],
                    sem0).wait()
            return carry

        lax.fori_loop(0, per_s // 16, w_body, 0)

    return k(table3, idx2)


def kernel(z, phoneme_mask, embed):
    B, N, Dz = z.shape
    rows = B * N
    flat = z.reshape(rows, Dz)
    mask2d = phoneme_mask.reshape(rows // 128, 128).astype(jnp.int32)
    idxg, idxo, loss = _tc_stage(flat, mask2d, embed)
    table = jnp.concatenate([embed, jnp.zeros((1, Dz), jnp.float32)], axis=0)
    table3 = table.reshape(K + 1, 2, Dz // 2).transpose(1, 0, 2).reshape(2, -1)
    idx2 = idxg.reshape(16, rows // 16)
    quant = _sc_gather(table3, idx2, rows)
    quantize = quant.reshape(B, N, Dz)
    indices = idxo.reshape(B, N)
    commit_loss = loss[0, 0] * (CW / (rows * Dz))
    return (quantize, indices, commit_loss)
